# baseline (speedup 1.0000x reference)
import functools

import jax
import jax.numpy as jnp
from jax import lax
from jax.experimental import pallas as pl
from jax.experimental.pallas import tpu as pltpu
from jax.experimental.pallas import tpu_sc as plsc

LN_EPS = 1e-5
OUT_F = 256
HALF_F = OUT_F // 2
ROW_BLK = 4096


def _round_bf16_bits(u):
    lsb = lax.shift_right_logical(u, 16) & 1
    rounded = u + 0x7FFF + lsb
    return rounded & jnp.int32(-65536)


def _fuse_body(emb_ref, w_ref, b_ref, g_ref, beta_ref, out_ref):
    x = emb_ref[...]
    y = jnp.dot(x, w_ref[...], preferred_element_type=jnp.float32)
    y = y + b_ref[...]
    mean = jnp.mean(y, axis=-1, keepdims=True)
    c = y - mean
    var = jnp.mean(c * c, axis=-1, keepdims=True)
    xhat = c * lax.rsqrt(var + LN_EPS)
    z = xhat * g_ref[...] + beta_ref[...]
    a = lax.bitcast_convert_type(z[:, :HALF_F], jnp.int32)
    b = lax.bitcast_convert_type(z[:, HALF_F:], jnp.int32)
    lo = lax.shift_right_logical(_round_bf16_bits(a), 16)
    hi = _round_bf16_bits(b)
    out_ref[...] = lo | hi


def _fused_table(emb_table, W, b, ln_gamma, ln_beta):
    vocab, emb_dim = emb_table.shape
    grid = (vocab + ROW_BLK - 1) // ROW_BLK
    return pl.pallas_call(
        _fuse_body,
        grid=(grid,),
        in_specs=[
            pl.BlockSpec((ROW_BLK, emb_dim), lambda i: (i, 0)),
            pl.BlockSpec((emb_dim, OUT_F), lambda i: (0, 0)),
            pl.BlockSpec((1, OUT_F), lambda i: (0, 0)),
            pl.BlockSpec((1, OUT_F), lambda i: (0, 0)),
            pl.BlockSpec((1, OUT_F), lambda i: (0, 0)),
        ],
        out_specs=pl.BlockSpec((ROW_BLK, HALF_F), lambda i: (i, 0)),
        out_shape=jax.ShapeDtypeStruct((vocab, HALF_F), jnp.int32),
    )(emb_table, W, b.reshape(1, OUT_F), ln_gamma.reshape(1, OUT_F),
      ln_beta.reshape(1, OUT_F))


_CHUNK = 128
_L = 16


def _sc_gather(table, ids):
    info = plsc.get_sparse_core_info()
    nw = info.num_cores * info.num_subcores
    n = ids.shape[0]
    assert n % (nw * _CHUNK) == 0
    b_per_w = n // nw
    n_chunks = b_per_w // _CHUNK
    mesh = plsc.VectorSubcoreMesh(core_axis_name="c", subcore_axis_name="s")

    @functools.partial(
        pl.kernel,
        mesh=mesh,
        out_type=jax.ShapeDtypeStruct((n, OUT_F), jnp.int32),
        scratch_types=(
            [pltpu.VMEM((b_per_w,), jnp.int32)]
            + [pltpu.VMEM((_CHUNK, HALF_F), jnp.int32)] * 2
            + [pltpu.VMEM((_CHUNK, OUT_F), jnp.int32)] * 2
            + [pltpu.SemaphoreType.DMA] * 4
        ),
    )
    def gather(table_hbm, idx_hbm, out_hbm, idx_v, pb0, pb1, ub0, ub1,
               gsem0, gsem1, ssem0, ssem1):
        pbufs = (pb0, pb1)
        ubufs = (ub0, ub1)
        gsems = (gsem0, gsem1)
        ssems = (ssem0, ssem1)
        wid = lax.axis_index("s") * info.num_cores + lax.axis_index("c")
        base = wid * b_per_w
        pltpu.sync_copy(idx_hbm.at[pl.ds(base, b_per_w)], idx_v)

        def start_gather(j):
            return pltpu.async_copy(
                table_hbm.at[idx_v.at[pl.ds(j * _CHUNK, _CHUNK)]],
                pbufs[j & 1], gsems[j & 1])

        def unpack(pb, ub):
            def row(r, carry):
                for p in range(HALF_F // _L):
                    w = pb[r, pl.ds(p * _L, _L)]
                    ub[r, pl.ds(p * _L, _L)] = lax.shift_left(w, 16)
                    ub[r, pl.ds(HALF_F + p * _L, _L)] = w & jnp.int32(-65536)
                return carry

            lax.fori_loop(0, _CHUNK, row, 0)

        g = {}
        s = {}
        g[0] = start_gather(0)
        for j in range(n_chunks):
            cur = j & 1
            g[j].wait()
            if j + 1 < n_chunks:
                g[j + 1] = start_gather(j + 1)
            if j >= 2:
                s[j - 2].wait()
            unpack(pbufs[cur], ubufs[cur])
            s[j] = pltpu.async_copy(
                ubufs[cur], out_hbm.at[pl.ds(base + j * _CHUNK, _CHUNK)],
                ssems[cur])
        for j in range(max(0, n_chunks - 2), n_chunks):
            s[j].wait()

    return gather(table, ids)


def kernel(input_ids, attention_mask, emb_table, W, b, ln_gamma, ln_beta):
    fused = _fused_table(emb_table, W, b, ln_gamma, ln_beta)
    bsz, seq = input_ids.shape
    ids = input_ids.reshape(-1).astype(jnp.int32)
    rows = lax.bitcast_convert_type(_sc_gather(fused, ids), jnp.float32)
    return rows.reshape(bsz, seq, OUT_F), attention_mask


# speedup vs baseline: 1.6374x; 1.6374x over previous
import functools

import jax
import jax.numpy as jnp
from jax import lax
from jax.experimental import pallas as pl
from jax.experimental.pallas import tpu as pltpu
from jax.experimental.pallas import tpu_sc as plsc

LN_EPS = 1e-5
OUT_F = 256
ROW_BLK = 4096


def _fuse_body(emb_ref, w_ref, b_ref, g_ref, beta_ref, out_ref):
    x = emb_ref[...]
    y = jnp.dot(x, w_ref[...], preferred_element_type=jnp.float32)
    y = y + b_ref[...]
    mean = jnp.mean(y, axis=-1, keepdims=True)
    c = y - mean
    var = jnp.mean(c * c, axis=-1, keepdims=True)
    xhat = c * lax.rsqrt(var + LN_EPS)
    out_ref[...] = xhat * g_ref[...] + beta_ref[...]


def _fused_table(emb_table, W, b, ln_gamma, ln_beta):
    vocab, emb_dim = emb_table.shape
    grid = (vocab + ROW_BLK - 1) // ROW_BLK
    return pl.pallas_call(
        _fuse_body,
        grid=(grid,),
        in_specs=[
            pl.BlockSpec((ROW_BLK, emb_dim), lambda i: (i, 0)),
            pl.BlockSpec((emb_dim, OUT_F), lambda i: (0, 0)),
            pl.BlockSpec((1, OUT_F), lambda i: (0, 0)),
            pl.BlockSpec((1, OUT_F), lambda i: (0, 0)),
            pl.BlockSpec((1, OUT_F), lambda i: (0, 0)),
        ],
        out_specs=pl.BlockSpec((ROW_BLK, OUT_F), lambda i: (i, 0)),
        out_shape=jax.ShapeDtypeStruct((vocab, OUT_F), jnp.float32),
    )(emb_table, W, b.reshape(1, OUT_F), ln_gamma.reshape(1, OUT_F),
      ln_beta.reshape(1, OUT_F))


_CHUNK = 128


def _sc_gather(table, ids):
    info = plsc.get_sparse_core_info()
    nw = info.num_cores * info.num_subcores
    n = ids.shape[0]
    assert n % (nw * _CHUNK) == 0
    b_per_w = n // nw
    n_chunks = b_per_w // _CHUNK
    mesh = plsc.VectorSubcoreMesh(core_axis_name="c", subcore_axis_name="s")

    nbuf = 2

    @functools.partial(
        pl.kernel,
        mesh=mesh,
        out_type=jax.ShapeDtypeStruct((n, OUT_F), jnp.float32),
        scratch_types=(
            [pltpu.VMEM((b_per_w,), jnp.int32)]
            + [pltpu.VMEM((_CHUNK, OUT_F), jnp.float32)] * nbuf
            + [pltpu.SemaphoreType.DMA] * (2 * nbuf)
        ),
    )
    def gather(table_hbm, idx_hbm, out_hbm, idx_v, *scratch):
        bufs = scratch[:nbuf]
        gsems = scratch[nbuf:2 * nbuf]
        ssems = scratch[2 * nbuf:]
        wid = lax.axis_index("s") * info.num_cores + lax.axis_index("c")
        base = wid * b_per_w
        pltpu.sync_copy(idx_hbm.at[pl.ds(base, b_per_w)], idx_v)

        def start_gather(j):
            b = j % nbuf
            return pltpu.async_copy(
                table_hbm.at[idx_v.at[pl.ds(j * _CHUNK, _CHUNK)]],
                bufs[b], gsems[b])

        g = {}
        s = {}
        for j in range(min(nbuf, n_chunks)):
            g[j] = start_gather(j)
        for j in range(n_chunks):
            b = j % nbuf
            nj = j + 1
            if nbuf <= nj < n_chunks:
                s[nj - nbuf].wait()
                g[nj] = start_gather(nj)
            g[j].wait()
            s[j] = pltpu.async_copy(
                bufs[b], out_hbm.at[pl.ds(base + j * _CHUNK, _CHUNK)],
                ssems[b])
        for j in range(max(0, n_chunks - nbuf), n_chunks):
            s[j].wait()

    return gather(table, ids)


def kernel(input_ids, attention_mask, emb_table, W, b, ln_gamma, ln_beta):
    fused = _fused_table(emb_table, W, b, ln_gamma, ln_beta)
    bsz, seq = input_ids.shape
    ids = input_ids.reshape(-1).astype(jnp.int32)
    rows = _sc_gather(fused, ids)
    return rows.reshape(bsz, seq, OUT_F), attention_mask
